# baseline (speedup 1.0000x reference)
import functools

import jax
import jax.numpy as jnp
from jax import lax
from jax.experimental import pallas as pl
from jax.experimental.pallas import tpu as pltpu
from jax.experimental.pallas import tpu_sc as plsc

N = 10000
NP = 10240
E = 320000
NW = 32
EPT = E // NW
C = 40
NCHUNK = EPT // C
EPT2 = 10240
C2 = 128
NCHUNK2 = EPT2 // C2
RPT = NP // 16
ZB = 40
BLK = 1024
GRID = NP // BLK

@functools.cache
def _mesh():
    return plsc.VectorSubcoreMesh(core_axis_name="c", subcore_axis_name="s")


def _deg_body(src_hbm, dst_hbm, out_hbm, hs_v, hd_v, idx_v):
    c = lax.axis_index("c")
    s = lax.axis_index("s")
    wid = s * 2 + c
    zero16 = jnp.zeros((16,), jnp.float32)

    def zinit(i, _):
        hs_v[pl.ds(i * 16, 16)] = zero16
        hd_v[pl.ds(i * 16, 16)] = zero16
        return 0

    lax.fori_loop(0, NP // 16, zinit, 0)

    ones16 = jnp.ones((16,), jnp.float32)
    base = wid * EPT
    CH = 2000

    def run_hist(hist_ref, e_hbm):
        def chunk(j, _):
            pltpu.sync_copy(e_hbm.at[pl.ds(base + j * CH, CH)], idx_v)

            def upd(t, __):
                idxs = idx_v[pl.ds(t * 16, 16)]
                plsc.addupdate_scatter(hist_ref, [idxs], ones16)
                return 0

            lax.fori_loop(0, CH // 16, upd, 0)
            return 0

        lax.fori_loop(0, EPT // CH, chunk, 0)

    run_hist(hs_v, src_hbm)
    run_hist(hd_v, dst_hbm)
    pltpu.sync_copy(hs_v, out_hbm.at[wid, 0])
    pltpu.sync_copy(hd_v, out_hbm.at[wid, 1])


@functools.cache
def _deg_call():
    return pl.kernel(
        _deg_body,
        mesh=_mesh(),
        out_type=jax.ShapeDtypeStruct((NW, 2, NP), jnp.float32),
        scratch_types=[
            pltpu.VMEM((NP,), jnp.float32),
            pltpu.VMEM((NP,), jnp.float32),
            pltpu.VMEM((2000,), jnp.int32),
        ],
        compiler_params=pltpu.CompilerParams(needs_layout_passes=False),
    )


NB = 4


def _make_scatter(D, Cc, nchunk):
    KD = D // 16
    ngrp_end = (nchunk - 2) // NB

    def body(h_hbm, src_hbm, dst_hbm, out_hbm, acc_s, zb_v, si_v, di_v,
             r0, r1, r2, r3, g0, g1, g2, g3, s0, s1, s2, s3, is0, is1):
        c = lax.axis_index("c")
        s = lax.axis_index("s")
        wid = s * 2 + c
        zero16 = jnp.zeros((16,), jnp.float32)

        R = (r0, r1, r2, r3)
        G = (g0, g1, g2, g3)
        S = (s0, s1, s2, s3)

        def ga(i, b):
            pltpu.async_copy(h_hbm.at[si_v.at[i]], R[b], G[b])

        def ga_w(i, b):
            pltpu.make_async_copy(h_hbm.at[si_v.at[i]], R[b], G[b]).wait()

        def sc(i, b):
            pltpu.async_copy(R[b], acc_s.at[di_v.at[i]], S[b], add=True)

        def sc_w(b):
            pltpu.make_async_copy(R[b], acc_s.at[di_v.at[0]], S[b]).wait()

        icp_s = pltpu.async_copy(src_hbm.at[wid], si_v, is0)
        icp_d = pltpu.async_copy(dst_hbm.at[wid], di_v, is1)

        def zrow(i, _):
            def zcol(k, __):
                zb_v[i, pl.ds(k * 16, 16)] = zero16
                return 0

            lax.fori_loop(0, KD, zcol, 0)
            return 0

        lax.fori_loop(0, ZB, zrow, 0)

        def zcopy(j, _):
            pltpu.sync_copy(zb_v, acc_s.at[pl.ds(s * RPT + j * ZB, ZB)])
            return 0

        lax.fori_loop(0, RPT // ZB, zcopy, 0)
        icp_s.wait()
        icp_d.wait()
        plsc.subcore_barrier()

        def slot(i, si):
            b = si % NB
            ga_w(i, b)
            sc(i, b)
            if si >= 2:
                sc_w((b + 2) % NB)
            if si + 2 <= nchunk - 1:
                ga(i + 2, (b + 2) % NB)

        ga(0, 0)
        ga(1, 1)
        for si in range(NB):
            slot(si, si)

        def grp(g, _):
            i0 = g * NB
            for b in range(NB):
                slot(i0 + b, NB + b)
            return 0

        lax.fori_loop(1, ngrp_end, grp, 0)
        for si in range(ngrp_end * NB, nchunk):
            slot(si, si)
        sc_w((nchunk - 2) % NB)
        sc_w((nchunk - 1) % NB)
        plsc.subcore_barrier()
        pltpu.sync_copy(acc_s.at[pl.ds(s * RPT, RPT)],
                        out_hbm.at[c, pl.ds(s * RPT, RPT)])

    return pl.kernel(
        body,
        mesh=_mesh(),
        out_type=jax.ShapeDtypeStruct((2, NP, D), jnp.float32),
        scratch_types=[
            pltpu.VMEM_SHARED((NP, D), jnp.float32),
            pltpu.VMEM((ZB, D), jnp.float32),
            pltpu.VMEM((nchunk, Cc), jnp.int32),
            pltpu.VMEM((nchunk, Cc), jnp.int32),
            pltpu.VMEM((Cc, D), jnp.float32),
            pltpu.VMEM((Cc, D), jnp.float32),
            pltpu.VMEM((Cc, D), jnp.float32),
            pltpu.VMEM((Cc, D), jnp.float32),
            pltpu.SemaphoreType.DMA,
            pltpu.SemaphoreType.DMA,
            pltpu.SemaphoreType.DMA,
            pltpu.SemaphoreType.DMA,
            pltpu.SemaphoreType.DMA,
            pltpu.SemaphoreType.DMA,
            pltpu.SemaphoreType.DMA,
            pltpu.SemaphoreType.DMA,
            pltpu.SemaphoreType.DMA,
            pltpu.SemaphoreType.DMA,
        ],
        compiler_params=pltpu.CompilerParams(use_tc_tiling_on_sc=False),
    )


_scatter128 = functools.cache(lambda: _make_scatter(128, C, NCHUNK))
_scatter64 = functools.cache(lambda: _make_scatter(64, C2, NCHUNK2))


def _norms(deg_ref):
    deg = jnp.sum(deg_ref[...], axis=0)
    ns = lax.rsqrt(jnp.maximum(deg[0], 1.0))
    nd = lax.rsqrt(jnp.maximum(deg[1], 1.0))
    return ns, nd


def _tc_a_body(deg_ref, x_ref, w_ref, o_ref):
    ns, _ = _norms(deg_ref)
    xw = jnp.dot(x_ref[...], w_ref[...], preferred_element_type=jnp.float32)
    o_ref[...] = xw * ns[:, None]


def _tc_b_body(deg_ref, p_ref, b1_ref, w2_ref, o_ref):
    ns, nd = _norms(deg_ref)
    agg = p_ref[0] + p_ref[1]
    h1 = jnp.maximum(agg * nd[:, None] + b1_ref[...], 0.0)
    o_ref[...] = jnp.dot(h1 * ns[:, None], w2_ref[...],
                         preferred_element_type=jnp.float32)


def _tc_c_body(deg_ref, q_ref, b2_ref, wf1_ref, bf1_ref, wf2_ref, bf2_ref,
               out0_ref, hl_ref):
    _, nd = _norms(deg_ref)
    agg = q_ref[0] + q_ref[1]
    h_last = jnp.maximum(agg * nd[:, None] + b2_ref[...], 0.0)
    m = jnp.maximum(jnp.dot(h_last, wf1_ref[...],
                            preferred_element_type=jnp.float32) + bf1_ref[...], 0.0)
    out0_ref[...] = jnp.dot(m, wf2_ref[...],
                            preferred_element_type=jnp.float32) + bf2_ref[...]
    hl_ref[...] = h_last


_DEG_SPEC = pl.BlockSpec((NW, 2, BLK), lambda i: (0, 0, i))


def _tc_a(deg_parts, x, W1):
    return pl.pallas_call(
        _tc_a_body,
        grid=(GRID,),
        in_specs=[
            _DEG_SPEC,
            pl.BlockSpec((BLK, 128), lambda i: (i, 0)),
            pl.BlockSpec((128, 128), lambda i: (0, 0)),
        ],
        out_specs=pl.BlockSpec((BLK, 128), lambda i: (i, 0)),
        out_shape=jax.ShapeDtypeStruct((NP, 128), jnp.float32),
    )(deg_parts, x, W1)


def _tc_b(deg_parts, p, b1, W2):
    return pl.pallas_call(
        _tc_b_body,
        grid=(GRID,),
        in_specs=[
            _DEG_SPEC,
            pl.BlockSpec((2, BLK, 128), lambda i: (0, i, 0)),
            pl.BlockSpec((1, 128), lambda i: (0, 0)),
            pl.BlockSpec((128, 64), lambda i: (0, 0)),
        ],
        out_specs=pl.BlockSpec((BLK, 64), lambda i: (i, 0)),
        out_shape=jax.ShapeDtypeStruct((NP, 64), jnp.float32),
    )(deg_parts, p, b1, W2)


def _tc_c(deg_parts, q, b2, Wf1, bf1, Wf2, bf2):
    return pl.pallas_call(
        _tc_c_body,
        grid=(GRID,),
        in_specs=[
            _DEG_SPEC,
            pl.BlockSpec((2, BLK, 64), lambda i: (0, i, 0)),
            pl.BlockSpec((1, 64), lambda i: (0, 0)),
            pl.BlockSpec((64, 32), lambda i: (0, 0)),
            pl.BlockSpec((1, 32), lambda i: (0, 0)),
            pl.BlockSpec((32, 64), lambda i: (0, 0)),
            pl.BlockSpec((1, 64), lambda i: (0, 0)),
        ],
        out_specs=[
            pl.BlockSpec((BLK, 64), lambda i: (i, 0)),
            pl.BlockSpec((BLK, 64), lambda i: (i, 0)),
        ],
        out_shape=[
            jax.ShapeDtypeStruct((N, 64), jnp.float32),
            jax.ShapeDtypeStruct((N, 64), jnp.float32),
        ],
    )(deg_parts, q, b2, Wf1, bf1, Wf2, bf2)


def kernel(x, edge_index, W1, b1, W2, b2, Wf1, bf1, Wf2, bf2):
    src = edge_index[0]
    dst = edge_index[1]
    src3 = src.reshape(NW, NCHUNK, C)
    dst3 = dst.reshape(NW, NCHUNK, C)
    pad = ((0, 0), (0, EPT2 - EPT))
    srcp = jnp.pad(src.reshape(NW, EPT), pad).reshape(NW, NCHUNK2, C2)
    dstp = jnp.pad(dst.reshape(NW, EPT), pad,
                   constant_values=NP - 1).reshape(NW, NCHUNK2, C2)
    deg_parts = _deg_call()(src, dst)
    h0 = _tc_a(deg_parts, x, W1)
    p = _scatter128()(h0, src3, dst3)
    h1w = _tc_b(deg_parts, p, b1.reshape(1, 128), W2)
    q = _scatter64()(h1w, srcp, dstp)
    out0, h_last = _tc_c(deg_parts, q, b2.reshape(1, 64),
                         Wf1, bf1.reshape(1, 32), Wf2, bf2.reshape(1, 64))
    return (out0, h_last)


# speedup vs baseline: 1.4356x; 1.4356x over previous
import functools

import jax
import jax.numpy as jnp
from jax import lax
from jax.experimental import pallas as pl
from jax.experimental.pallas import tpu as pltpu
from jax.experimental.pallas import tpu_sc as plsc

N = 10000
NP = 10240
E = 320000
NW = 32
EPT = E // NW
C = 40
NCHUNK = EPT // C
C2 = 80
NCHUNK2 = EPT // C2
RPT = NP // 16
ZB = 40
BLK = 1024
GRID = NP // BLK

@functools.cache
def _mesh():
    return plsc.VectorSubcoreMesh(core_axis_name="c", subcore_axis_name="s")


def _deg_body(src_hbm, dst_hbm, out_hbm, hs_v, hd_v, idx_v):
    c = lax.axis_index("c")
    s = lax.axis_index("s")
    wid = s * 2 + c
    zero16 = jnp.zeros((16,), jnp.float32)

    def zinit(i, _):
        hs_v[pl.ds(i * 16, 16)] = zero16
        hd_v[pl.ds(i * 16, 16)] = zero16
        return 0

    lax.fori_loop(0, NP // 16, zinit, 0)

    ones16 = jnp.ones((16,), jnp.float32)
    base = wid * EPT
    CH = 2000

    def run_hist(hist_ref, e_hbm):
        def chunk(j, _):
            pltpu.sync_copy(e_hbm.at[pl.ds(base + j * CH, CH)], idx_v)

            def upd(t, __):
                idxs = idx_v[pl.ds(t * 16, 16)]
                plsc.addupdate_scatter(hist_ref, [idxs], ones16)
                return 0

            lax.fori_loop(0, CH // 16, upd, 0)
            return 0

        lax.fori_loop(0, EPT // CH, chunk, 0)

    run_hist(hs_v, src_hbm)
    run_hist(hd_v, dst_hbm)
    pltpu.sync_copy(hs_v, out_hbm.at[wid, 0])
    pltpu.sync_copy(hd_v, out_hbm.at[wid, 1])


@functools.cache
def _deg_call():
    return pl.kernel(
        _deg_body,
        mesh=_mesh(),
        out_type=jax.ShapeDtypeStruct((NW, 2, NP), jnp.float32),
        scratch_types=[
            pltpu.VMEM((NP,), jnp.float32),
            pltpu.VMEM((NP,), jnp.float32),
            pltpu.VMEM((2000,), jnp.int32),
        ],
        compiler_params=pltpu.CompilerParams(needs_layout_passes=False),
    )


NB = 4


def _make_scatter(D, Cc, nchunk):
    KD = D // 16
    ngrp_end = (nchunk - 2) // NB

    def body(h_hbm, src_hbm, dst_hbm, out_hbm, acc_s, zb_v, si_v, di_v,
             r0, r1, r2, r3, g0, g1, g2, g3, s0, s1, s2, s3, is0, is1):
        c = lax.axis_index("c")
        s = lax.axis_index("s")
        wid = s * 2 + c
        zero16 = jnp.zeros((16,), jnp.float32)

        R = (r0, r1, r2, r3)
        G = (g0, g1, g2, g3)
        S = (s0, s1, s2, s3)

        def ga(i, b):
            pltpu.async_copy(h_hbm.at[si_v.at[i]], R[b], G[b])

        def ga_w(i, b):
            pltpu.make_async_copy(h_hbm.at[si_v.at[i]], R[b], G[b]).wait()

        def sc(i, b):
            pltpu.async_copy(R[b], acc_s.at[di_v.at[i]], S[b], add=True)

        def sc_w(b):
            pltpu.make_async_copy(R[b], acc_s.at[di_v.at[0]], S[b]).wait()

        icp_s = pltpu.async_copy(src_hbm.at[wid], si_v, is0)
        icp_d = pltpu.async_copy(dst_hbm.at[wid], di_v, is1)

        def zrow(i, _):
            def zcol(k, __):
                zb_v[i, pl.ds(k * 16, 16)] = zero16
                return 0

            lax.fori_loop(0, KD, zcol, 0)
            return 0

        lax.fori_loop(0, ZB, zrow, 0)

        def zcopy(j, _):
            pltpu.sync_copy(zb_v, acc_s.at[pl.ds(s * RPT + j * ZB, ZB)])
            return 0

        lax.fori_loop(0, RPT // ZB, zcopy, 0)
        icp_s.wait()
        icp_d.wait()
        plsc.subcore_barrier()

        def slot(i, si):
            b = si % NB
            ga_w(i, b)
            sc(i, b)
            if si >= 2:
                sc_w((b + 2) % NB)
            if si + 2 <= nchunk - 1:
                ga(i + 2, (b + 2) % NB)

        ga(0, 0)
        ga(1, 1)
        for si in range(NB):
            slot(si, si)

        def grp(g, _):
            i0 = g * NB
            for b in range(NB):
                slot(i0 + b, NB + b)
            return 0

        lax.fori_loop(1, ngrp_end, grp, 0)
        for si in range(ngrp_end * NB, nchunk):
            slot(si, si)
        sc_w((nchunk - 2) % NB)
        sc_w((nchunk - 1) % NB)
        plsc.subcore_barrier()
        pltpu.sync_copy(acc_s.at[pl.ds(s * RPT, RPT)],
                        out_hbm.at[c, pl.ds(s * RPT, RPT)])

    return pl.kernel(
        body,
        mesh=_mesh(),
        out_type=jax.ShapeDtypeStruct((2, NP, D), jnp.float32),
        scratch_types=[
            pltpu.VMEM_SHARED((NP, D), jnp.float32),
            pltpu.VMEM((ZB, D), jnp.float32),
            pltpu.VMEM((nchunk, Cc), jnp.int32),
            pltpu.VMEM((nchunk, Cc), jnp.int32),
            pltpu.VMEM((Cc, D), jnp.float32),
            pltpu.VMEM((Cc, D), jnp.float32),
            pltpu.VMEM((Cc, D), jnp.float32),
            pltpu.VMEM((Cc, D), jnp.float32),
            pltpu.SemaphoreType.DMA,
            pltpu.SemaphoreType.DMA,
            pltpu.SemaphoreType.DMA,
            pltpu.SemaphoreType.DMA,
            pltpu.SemaphoreType.DMA,
            pltpu.SemaphoreType.DMA,
            pltpu.SemaphoreType.DMA,
            pltpu.SemaphoreType.DMA,
            pltpu.SemaphoreType.DMA,
            pltpu.SemaphoreType.DMA,
        ],
        compiler_params=pltpu.CompilerParams(use_tc_tiling_on_sc=False),
    )


_scatter128 = functools.cache(lambda: _make_scatter(128, C, NCHUNK))
_scatter64 = functools.cache(lambda: _make_scatter(64, C2, NCHUNK2))


def _norms(deg_ref):
    deg = jnp.sum(deg_ref[...], axis=0)
    ns = lax.rsqrt(jnp.maximum(deg[0], 1.0))
    nd = lax.rsqrt(jnp.maximum(deg[1], 1.0))
    return ns, nd


def _tc_a_body(deg_ref, x_ref, w_ref, o_ref):
    ns, _ = _norms(deg_ref)
    xw = jnp.dot(x_ref[...], w_ref[...], preferred_element_type=jnp.float32)
    o_ref[...] = xw * ns[:, None]


def _tc_b_body(deg_ref, p_ref, b1_ref, w2_ref, o_ref):
    ns, nd = _norms(deg_ref)
    agg = p_ref[0] + p_ref[1]
    h1 = jnp.maximum(agg * nd[:, None] + b1_ref[...], 0.0)
    o_ref[...] = jnp.dot(h1 * ns[:, None], w2_ref[...],
                         preferred_element_type=jnp.float32)


def _tc_c_body(deg_ref, q_ref, b2_ref, wf1_ref, bf1_ref, wf2_ref, bf2_ref,
               out0_ref, hl_ref):
    _, nd = _norms(deg_ref)
    agg = q_ref[0] + q_ref[1]
    h_last = jnp.maximum(agg * nd[:, None] + b2_ref[...], 0.0)
    m = jnp.maximum(jnp.dot(h_last, wf1_ref[...],
                            preferred_element_type=jnp.float32) + bf1_ref[...], 0.0)
    out0_ref[...] = jnp.dot(m, wf2_ref[...],
                            preferred_element_type=jnp.float32) + bf2_ref[...]
    hl_ref[...] = h_last


_DEG_SPEC = pl.BlockSpec((NW, 2, BLK), lambda i: (0, 0, i))


def _tc_a(deg_parts, x, W1):
    return pl.pallas_call(
        _tc_a_body,
        grid=(GRID,),
        in_specs=[
            _DEG_SPEC,
            pl.BlockSpec((BLK, 128), lambda i: (i, 0)),
            pl.BlockSpec((128, 128), lambda i: (0, 0)),
        ],
        out_specs=pl.BlockSpec((BLK, 128), lambda i: (i, 0)),
        out_shape=jax.ShapeDtypeStruct((NP, 128), jnp.float32),
    )(deg_parts, x, W1)


def _tc_b(deg_parts, p, b1, W2):
    return pl.pallas_call(
        _tc_b_body,
        grid=(GRID,),
        in_specs=[
            _DEG_SPEC,
            pl.BlockSpec((2, BLK, 128), lambda i: (0, i, 0)),
            pl.BlockSpec((1, 128), lambda i: (0, 0)),
            pl.BlockSpec((128, 64), lambda i: (0, 0)),
        ],
        out_specs=pl.BlockSpec((BLK, 64), lambda i: (i, 0)),
        out_shape=jax.ShapeDtypeStruct((NP, 64), jnp.float32),
    )(deg_parts, p, b1, W2)


def _tc_c(deg_parts, q, b2, Wf1, bf1, Wf2, bf2):
    return pl.pallas_call(
        _tc_c_body,
        grid=(GRID,),
        in_specs=[
            _DEG_SPEC,
            pl.BlockSpec((2, BLK, 64), lambda i: (0, i, 0)),
            pl.BlockSpec((1, 64), lambda i: (0, 0)),
            pl.BlockSpec((64, 32), lambda i: (0, 0)),
            pl.BlockSpec((1, 32), lambda i: (0, 0)),
            pl.BlockSpec((32, 64), lambda i: (0, 0)),
            pl.BlockSpec((1, 64), lambda i: (0, 0)),
        ],
        out_specs=[
            pl.BlockSpec((BLK, 64), lambda i: (i, 0)),
            pl.BlockSpec((BLK, 64), lambda i: (i, 0)),
        ],
        out_shape=[
            jax.ShapeDtypeStruct((N, 64), jnp.float32),
            jax.ShapeDtypeStruct((N, 64), jnp.float32),
        ],
    )(deg_parts, q, b2, Wf1, bf1, Wf2, bf2)


def kernel(x, edge_index, W1, b1, W2, b2, Wf1, bf1, Wf2, bf2):
    src = edge_index[0]
    dst = edge_index[1]
    src3 = src.reshape(NW, NCHUNK, C)
    dst3 = dst.reshape(NW, NCHUNK, C)
    srcp = src.reshape(NW, NCHUNK2, C2)
    dstp = dst.reshape(NW, NCHUNK2, C2)
    deg_parts = _deg_call()(src, dst)
    h0 = _tc_a(deg_parts, x, W1)
    p = _scatter128()(h0, src3, dst3)
    h1w = _tc_b(deg_parts, p, b1.reshape(1, 128), W2)
    q = _scatter64()(h1w, srcp, dstp)
    out0, h_last = _tc_c(deg_parts, q, b2.reshape(1, 64),
                         Wf1, bf1.reshape(1, 32), Wf2, bf2.reshape(1, 64))
    return (out0, h_last)
